# baseline (speedup 1.0000x reference)
import functools

import jax
import jax.numpy as jnp
from jax.experimental import pallas as pl
from jax.experimental.pallas import tpu as pltpu
from jax.experimental.pallas import tpu_sc as plsc

NUM_EMBEDDINGS = 1024
EMBEDDING_DIM = 64
SPARSITY_LEVEL = 5
COMMITMENT_COST = 0.25
EPS = 1e-10
DIVERSITY_WEIGHT = 0.001

NH = 8
NL = 128

TILE_B = 2048
NUM_TOKENS = 16384

SC_CORES = 2
SC_SUBCORES = 16
SC_WORKERS = SC_CORES * SC_SUBCORES
SC_CHUNK = NUM_TOKENS // SC_WORKERS
SC_LANES = 16


def _omp_step_kernel(k, d_ref, dr_ref, res_ref, usage_prev_ref, prev_idx_ref,
                     res_out_ref, idx_out_ref, alpha_out_ref, usage_out_ref):
    t = pl.program_id(0)
    d_raw = d_ref[...]
    nrm = jnp.sqrt(jnp.sum(d_raw * d_raw, axis=0, keepdims=True))
    dn = d_raw / (nrm + EPS)
    res = res_ref[...]
    bsz = res.shape[1]

    corr = jax.lax.dot_general(
        dn, res, (((0,), (0,)), ((), ())),
        preferred_element_type=jnp.float32)
    v = jnp.abs(corr).reshape(NH, NL, bsz)

    if k > 0:
        usage = usage_prev_ref[...]
        avg = jnp.sum(usage) / NUM_EMBEDDINGS
        bonus = DIVERSITY_WEIGHT * jnp.maximum(avg - usage, 0.0)
        v = v + bonus[:, :, None]

    i0 = jax.lax.broadcasted_iota(jnp.int32, (NH, NL, bsz), 0)
    i1 = jax.lax.broadcasted_iota(jnp.int32, (NH, NL, bsz), 1)
    ii = i0 * NL + i1
    for j in range(k):
        pj = prev_idx_ref[j, :]
        v = jnp.where(ii == pj[None, None, :], 0.0, v)

    l_per_slab = jnp.argmax(v, axis=1)
    m_per_slab = jnp.max(v, axis=1)
    h_star = jnp.argmax(m_per_slab, axis=0)
    ih = jax.lax.broadcasted_iota(jnp.int32, (NH, bsz), 0)
    oh_h = (ih == h_star[None, :]).astype(jnp.float32)
    l_star = jnp.sum(
        l_per_slab.astype(jnp.float32) * oh_h, axis=0).astype(jnp.int32)
    idx = h_star * NL + l_star

    il = jax.lax.broadcasted_iota(jnp.int32, (NL, bsz), 0)
    oh_l = (il == l_star[None, :]).astype(jnp.float32)

    dnr = (dr_ref[...] / (nrm.reshape(NH, 1, NL) + EPS)
           ).reshape(NH * EMBEDDING_DIM, NL)
    c0 = dnr.astype(jnp.bfloat16)
    r1 = dnr - c0.astype(jnp.float32)
    c1 = r1.astype(jnp.bfloat16)
    c2 = (r1 - c1.astype(jnp.float32)).astype(jnp.bfloat16)
    oh_l_b = oh_l.astype(jnp.bfloat16)
    dims = (((1,), (0,)), ((), ()))
    u = (jax.lax.dot_general(c0, oh_l_b, dims,
                             preferred_element_type=jnp.float32)
         + jax.lax.dot_general(c1, oh_l_b, dims,
                               preferred_element_type=jnp.float32)
         + jax.lax.dot_general(c2, oh_l_b, dims,
                               preferred_element_type=jnp.float32))
    d_sel = jnp.sum(
        u.reshape(NH, EMBEDDING_DIM, bsz) * oh_h[:, None, :], axis=0)

    num = jnp.sum(res * d_sel, axis=0, keepdims=True)
    den = jnp.sum(d_sel * d_sel, axis=0, keepdims=True)
    alpha = num / (den + EPS)

    res_out_ref[...] = res - d_sel * alpha
    idx_out_ref[...] = idx[None, :]
    alpha_out_ref[...] = alpha

    hist = jax.lax.dot_general(
        oh_h, oh_l, (((1,), (1,)), ((), ())),
        precision=jax.lax.Precision.HIGHEST,
        preferred_element_type=jnp.float32)

    @pl.when(t == 0)
    def _init():
        if k > 0:
            usage_out_ref[...] = usage_prev_ref[...] + hist
        else:
            usage_out_ref[...] = hist

    @pl.when(t != 0)
    def _acc():
        usage_out_ref[...] += hist


def _omp_step(k, d_raw, d_raw_r, res, usage_prev, prev_idx):
    b = res.shape[1]
    grid = (b // TILE_B,)
    in_specs = [
        pl.BlockSpec((EMBEDDING_DIM, NUM_EMBEDDINGS), lambda t: (0, 0)),
        pl.BlockSpec((NH, EMBEDDING_DIM, NL), lambda t: (0, 0, 0)),
        pl.BlockSpec((EMBEDDING_DIM, TILE_B), lambda t: (0, t)),
    ]
    args = [d_raw, d_raw_r, res]
    if k > 0:
        in_specs.append(pl.BlockSpec((NH, NL), lambda t: (0, 0)))
        in_specs.append(pl.BlockSpec((k, TILE_B), lambda t: (0, t)))
        args.append(usage_prev)
        args.append(prev_idx)
        body = functools.partial(_omp_step_kernel, k)
    else:
        def body(d_ref, dr_ref, res_ref, *out_refs):
            _omp_step_kernel(0, d_ref, dr_ref, res_ref, None, None, *out_refs)

    out_shape = [
        jax.ShapeDtypeStruct((EMBEDDING_DIM, b), jnp.float32),
        jax.ShapeDtypeStruct((1, b), jnp.int32),
        jax.ShapeDtypeStruct((1, b), jnp.float32),
        jax.ShapeDtypeStruct((NH, NL), jnp.float32),
    ]
    out_specs = [
        pl.BlockSpec((EMBEDDING_DIM, TILE_B), lambda t: (0, t)),
        pl.BlockSpec((1, TILE_B), lambda t: (0, t)),
        pl.BlockSpec((1, TILE_B), lambda t: (0, t)),
        pl.BlockSpec((NH, NL), lambda t: (0, 0)),
    ]
    return pl.pallas_call(
        body,
        grid=grid,
        in_specs=in_specs,
        out_specs=out_specs,
        out_shape=out_shape,
    )(*args)


def _sc_scatter_body(idx_hbm, alpha_hbm, coef_hbm, *scratch):
    idx_vs = scratch[0:SPARSITY_LEVEL]
    alpha_vs = scratch[SPARSITY_LEVEL:2 * SPARSITY_LEVEL]
    off_vs = scratch[2 * SPARSITY_LEVEL:3 * SPARSITY_LEVEL]
    sem = scratch[3 * SPARSITY_LEVEL]
    wid = jax.lax.axis_index("s") * SC_CORES + jax.lax.axis_index("c")
    base = wid * SC_CHUNK
    lane = jax.lax.iota(jnp.int32, SC_LANES)
    for r in range(SPARSITY_LEVEL):
        pltpu.sync_copy(idx_hbm.at[pl.ds(r * NUM_TOKENS + base, SC_CHUNK)],
                        idx_vs[r])
        pltpu.sync_copy(alpha_hbm.at[pl.ds(r * NUM_TOKENS + base, SC_CHUNK)],
                        alpha_vs[r])
        for i in range(SC_CHUNK // SC_LANES):
            sl = pl.ds(i * SC_LANES, SC_LANES)
            off_vs[r][sl] = (idx_vs[r][sl] * NUM_TOKENS
                             + (base + i * SC_LANES) + lane)
    descs = [pltpu.async_copy(alpha_vs[r], coef_hbm.at[off_vs[r]], sem)
             for r in range(SPARSITY_LEVEL)]
    for d in descs:
        d.wait()


_sc_scatter = pl.kernel(
    _sc_scatter_body,
    out_type=(),
    mesh=plsc.VectorSubcoreMesh(core_axis_name="c", subcore_axis_name="s"),
    scratch_types=(
        [pltpu.VMEM((SC_CHUNK,), jnp.int32) for _ in range(SPARSITY_LEVEL)]
        + [pltpu.VMEM((SC_CHUNK,), jnp.float32)
           for _ in range(SPARSITY_LEVEL)]
        + [pltpu.VMEM((SC_CHUNK,), jnp.int32) for _ in range(SPARSITY_LEVEL)]
        + [pltpu.SemaphoreType.DMA]
    ),
)


def _epilogue_kernel(x_ref, res_ref, out_ref, loss_ref):
    t = pl.program_id(0)
    x = x_ref[...]
    z_dl = x - res_ref[...]
    delta = z_dl - x
    out_ref[...] = x + delta

    part = jnp.sum(delta * delta).reshape(1, 1)

    @pl.when(t == 0)
    def _init():
        loss_ref[...] = part

    @pl.when(t != 0)
    def _acc():
        loss_ref[...] += part


def _epilogue(x, res):
    b = x.shape[1]
    grid = (b // TILE_B,)
    out_shape = [
        jax.ShapeDtypeStruct((EMBEDDING_DIM, b), jnp.float32),
        jax.ShapeDtypeStruct((1, 1), jnp.float32),
    ]
    return pl.pallas_call(
        _epilogue_kernel,
        grid=grid,
        in_specs=[
            pl.BlockSpec((EMBEDDING_DIM, TILE_B), lambda t: (0, t)),
            pl.BlockSpec((EMBEDDING_DIM, TILE_B), lambda t: (0, t)),
        ],
        out_specs=[
            pl.BlockSpec((EMBEDDING_DIM, TILE_B), lambda t: (0, t)),
            pl.BlockSpec((1, 1), lambda t: (0, 0)),
        ],
        out_shape=out_shape,
    )(x, res)


def kernel(z_e, dictionary):
    n, c, h, w = z_e.shape
    z = jnp.transpose(z_e, (0, 2, 3, 1))
    x = z.reshape(-1, EMBEDDING_DIM).T
    b = x.shape[1]

    d_raw_r = dictionary.reshape(EMBEDDING_DIM, NH, NL).transpose(1, 0, 2)

    coef_ref = jax.new_ref(jnp.zeros((NUM_EMBEDDINGS * b,), jnp.float32))

    res = x
    usage = None
    idx_list = []
    alpha_list = []
    for k in range(SPARSITY_LEVEL):
        prev_idx = jnp.concatenate(idx_list, axis=0) if k > 0 else None
        res, idx_k, alpha_k, usage = _omp_step(k, dictionary, d_raw_r, res,
                                               usage, prev_idx)
        idx_list.append(idx_k)
        alpha_list.append(alpha_k)

    idx_hist = jnp.concatenate(idx_list, axis=0).reshape(-1)
    alpha_hist = jnp.concatenate(alpha_list, axis=0).reshape(-1)
    _sc_scatter(idx_hist, alpha_hist, coef_ref)

    z_dl_st_flat, loss_sum = _epilogue(x, res)

    coef = coef_ref[...].reshape(NUM_EMBEDDINGS, b)

    m = loss_sum[0, 0] / (n * h * w * EMBEDDING_DIM)
    loss = COMMITMENT_COST * m + m

    out1 = z_dl_st_flat.T.reshape(n, h, w, c).transpose(0, 3, 1, 2)
    return (out1, loss, coef)


# speedup vs baseline: 1.4757x; 1.4757x over previous
import functools

import jax
import jax.numpy as jnp
from jax.experimental import pallas as pl

NUM_EMBEDDINGS = 1024
EMBEDDING_DIM = 64
SPARSITY_LEVEL = 5
COMMITMENT_COST = 0.25
EPS = 1e-10
DIVERSITY_WEIGHT = 0.001

NH = 8
NL = 128

TILE_B = 2048


def _omp_step_kernel(k, d_ref, dr_ref, res_ref, usage_prev_ref, prev_idx_ref,
                     res_out_ref, idx_out_ref, alpha_out_ref, usage_out_ref):
    t = pl.program_id(0)
    d_raw = d_ref[...]
    nrm = jnp.sqrt(jnp.sum(d_raw * d_raw, axis=0, keepdims=True))
    dn = d_raw / (nrm + EPS)
    res = res_ref[...]
    bsz = res.shape[1]

    corr = jax.lax.dot_general(
        dn, res, (((0,), (0,)), ((), ())),
        preferred_element_type=jnp.float32)
    v = jnp.abs(corr).reshape(NH, NL, bsz)

    if k > 0:
        usage = usage_prev_ref[...]
        avg = jnp.sum(usage) / NUM_EMBEDDINGS
        bonus = DIVERSITY_WEIGHT * jnp.maximum(avg - usage, 0.0)
        v = v + bonus[:, :, None]

    i0 = jax.lax.broadcasted_iota(jnp.int32, (NH, NL, bsz), 0)
    i1 = jax.lax.broadcasted_iota(jnp.int32, (NH, NL, bsz), 1)
    ii = i0 * NL + i1
    for j in range(k):
        pj = prev_idx_ref[j, :]
        v = jnp.where(ii == pj[None, None, :], 0.0, v)

    l_per_slab = jnp.argmax(v, axis=1)
    m_per_slab = jnp.max(v, axis=1)
    h_star = jnp.argmax(m_per_slab, axis=0)
    ih = jax.lax.broadcasted_iota(jnp.int32, (NH, bsz), 0)
    oh_h = (ih == h_star[None, :]).astype(jnp.float32)
    l_star = jnp.sum(
        l_per_slab.astype(jnp.float32) * oh_h, axis=0).astype(jnp.int32)
    idx = h_star * NL + l_star

    il = jax.lax.broadcasted_iota(jnp.int32, (NL, bsz), 0)
    oh_l = (il == l_star[None, :]).astype(jnp.float32)

    dnr = (dr_ref[...] / (nrm.reshape(NH, 1, NL) + EPS)
           ).reshape(NH * EMBEDDING_DIM, NL)
    c0 = dnr.astype(jnp.bfloat16)
    r1 = dnr - c0.astype(jnp.float32)
    c1 = r1.astype(jnp.bfloat16)
    c2 = (r1 - c1.astype(jnp.float32)).astype(jnp.bfloat16)
    oh_l_b = oh_l.astype(jnp.bfloat16)
    dims = (((1,), (0,)), ((), ()))
    u = (jax.lax.dot_general(c0, oh_l_b, dims,
                             preferred_element_type=jnp.float32)
         + jax.lax.dot_general(c1, oh_l_b, dims,
                               preferred_element_type=jnp.float32)
         + jax.lax.dot_general(c2, oh_l_b, dims,
                               preferred_element_type=jnp.float32))
    d_sel = jnp.sum(
        u.reshape(NH, EMBEDDING_DIM, bsz) * oh_h[:, None, :], axis=0)

    num = jnp.sum(res * d_sel, axis=0, keepdims=True)
    den = jnp.sum(d_sel * d_sel, axis=0, keepdims=True)
    alpha = num / (den + EPS)

    res_out_ref[...] = res - d_sel * alpha
    idx_out_ref[...] = idx[None, :]
    alpha_out_ref[...] = alpha

    hist = jax.lax.dot_general(
        oh_h, oh_l, (((1,), (1,)), ((), ())),
        precision=jax.lax.Precision.HIGHEST,
        preferred_element_type=jnp.float32)

    @pl.when(t == 0)
    def _init():
        if k > 0:
            usage_out_ref[...] = usage_prev_ref[...] + hist
        else:
            usage_out_ref[...] = hist

    @pl.when(t != 0)
    def _acc():
        usage_out_ref[...] += hist


def _omp_step(k, d_raw, d_raw_r, res, usage_prev, prev_idx):
    b = res.shape[1]
    grid = (b // TILE_B,)
    in_specs = [
        pl.BlockSpec((EMBEDDING_DIM, NUM_EMBEDDINGS), lambda t: (0, 0)),
        pl.BlockSpec((NH, EMBEDDING_DIM, NL), lambda t: (0, 0, 0)),
        pl.BlockSpec((EMBEDDING_DIM, TILE_B), lambda t: (0, t)),
    ]
    args = [d_raw, d_raw_r, res]
    if k > 0:
        in_specs.append(pl.BlockSpec((NH, NL), lambda t: (0, 0)))
        in_specs.append(pl.BlockSpec((k, TILE_B), lambda t: (0, t)))
        args.append(usage_prev)
        args.append(prev_idx)
        body = functools.partial(_omp_step_kernel, k)
    else:
        def body(d_ref, dr_ref, res_ref, *out_refs):
            _omp_step_kernel(0, d_ref, dr_ref, res_ref, None, None, *out_refs)

    out_shape = [
        jax.ShapeDtypeStruct((EMBEDDING_DIM, b), jnp.float32),
        jax.ShapeDtypeStruct((1, b), jnp.int32),
        jax.ShapeDtypeStruct((1, b), jnp.float32),
        jax.ShapeDtypeStruct((NH, NL), jnp.float32),
    ]
    out_specs = [
        pl.BlockSpec((EMBEDDING_DIM, TILE_B), lambda t: (0, t)),
        pl.BlockSpec((1, TILE_B), lambda t: (0, t)),
        pl.BlockSpec((1, TILE_B), lambda t: (0, t)),
        pl.BlockSpec((NH, NL), lambda t: (0, 0)),
    ]
    return pl.pallas_call(
        body,
        grid=grid,
        in_specs=in_specs,
        out_specs=out_specs,
        out_shape=out_shape,
    )(*args)


def _finalize_kernel(x_ref, res_ref, idx_ref, alpha_ref,
                     out_ref, coef_ref, loss_ref):
    t = pl.program_id(0)
    x = x_ref[...]
    bsz = x.shape[1]

    ii = jax.lax.broadcasted_iota(jnp.int32, (NUM_EMBEDDINGS, bsz), 0)
    coef = jnp.zeros((NUM_EMBEDDINGS, bsz), jnp.float32)
    for j in range(SPARSITY_LEVEL):
        sel = ii == idx_ref[j, :][None, :]
        coef = jnp.where(sel, alpha_ref[j, :][None, :], coef)
    coef_ref[...] = coef

    z_dl = x - res_ref[...]
    delta = z_dl - x
    out_ref[...] = x + delta

    part = jnp.sum(delta * delta).reshape(1, 1)

    @pl.when(t == 0)
    def _init():
        loss_ref[...] = part

    @pl.when(t != 0)
    def _acc():
        loss_ref[...] += part


def _finalize(x, res, idx_hist, alpha_hist):
    b = x.shape[1]
    grid = (b // TILE_B,)
    out_shape = [
        jax.ShapeDtypeStruct((EMBEDDING_DIM, b), jnp.float32),
        jax.ShapeDtypeStruct((NUM_EMBEDDINGS, b), jnp.float32),
        jax.ShapeDtypeStruct((1, 1), jnp.float32),
    ]
    return pl.pallas_call(
        _finalize_kernel,
        grid=grid,
        in_specs=[
            pl.BlockSpec((EMBEDDING_DIM, TILE_B), lambda t: (0, t)),
            pl.BlockSpec((EMBEDDING_DIM, TILE_B), lambda t: (0, t)),
            pl.BlockSpec((SPARSITY_LEVEL, TILE_B), lambda t: (0, t)),
            pl.BlockSpec((SPARSITY_LEVEL, TILE_B), lambda t: (0, t)),
        ],
        out_specs=[
            pl.BlockSpec((EMBEDDING_DIM, TILE_B), lambda t: (0, t)),
            pl.BlockSpec((NUM_EMBEDDINGS, TILE_B), lambda t: (0, t)),
            pl.BlockSpec((1, 1), lambda t: (0, 0)),
        ],
        out_shape=out_shape,
    )(x, res, idx_hist, alpha_hist)


def kernel(z_e, dictionary):
    n, c, h, w = z_e.shape
    z = jnp.transpose(z_e, (0, 2, 3, 1))
    x = z.reshape(-1, EMBEDDING_DIM).T

    d_raw_r = dictionary.reshape(EMBEDDING_DIM, NH, NL).transpose(1, 0, 2)

    res = x
    usage = None
    idx_list = []
    alpha_list = []
    for k in range(SPARSITY_LEVEL):
        prev_idx = jnp.concatenate(idx_list, axis=0) if k > 0 else None
        res, idx_k, alpha_k, usage = _omp_step(k, dictionary, d_raw_r, res,
                                               usage, prev_idx)
        idx_list.append(idx_k)
        alpha_list.append(alpha_k)

    idx_hist = jnp.concatenate(idx_list, axis=0)
    alpha_hist = jnp.concatenate(alpha_list, axis=0)

    z_dl_st_flat, coef, loss_sum = _finalize(x, res, idx_hist, alpha_hist)

    m = loss_sum[0, 0] / (n * h * w * EMBEDDING_DIM)
    loss = COMMITMENT_COST * m + m

    out1 = z_dl_st_flat.T.reshape(n, h, w, c).transpose(0, 3, 1, 2)
    return (out1, loss, coef)


# speedup vs baseline: 1.5086x; 1.0223x over previous
import functools

import jax
import jax.numpy as jnp
from jax.experimental import pallas as pl

NUM_EMBEDDINGS = 1024
EMBEDDING_DIM = 64
SPARSITY_LEVEL = 5
COMMITMENT_COST = 0.25
EPS = 1e-10
DIVERSITY_WEIGHT = 0.001

NH = 8
NL = 128

TILE_B = 2048


def _omp_step_kernel(k, d_ref, dr_ref, res_ref, usage_prev_ref, prev_idx_ref,
                     res_out_ref, idx_out_ref, alpha_out_ref, usage_out_ref):
    t = pl.program_id(0)
    d_raw = d_ref[...]
    nrm = jnp.sqrt(jnp.sum(d_raw * d_raw, axis=0, keepdims=True))
    dn = d_raw / (nrm + EPS)
    res = res_ref[...]
    bsz = res.shape[1]

    corr = jax.lax.dot_general(
        dn, res, (((0,), (0,)), ((), ())),
        preferred_element_type=jnp.float32)
    v = jnp.abs(corr).reshape(NH, NL, bsz)

    if k > 0:
        usage = usage_prev_ref[...]
        avg = jnp.sum(usage) / NUM_EMBEDDINGS
        bonus = DIVERSITY_WEIGHT * jnp.maximum(avg - usage, 0.0)
        v = v + bonus[:, :, None]

    i0 = jax.lax.broadcasted_iota(jnp.int32, (NH, NL, bsz), 0)
    i1 = jax.lax.broadcasted_iota(jnp.int32, (NH, NL, bsz), 1)
    ii = i0 * NL + i1
    for j in range(k):
        pj = prev_idx_ref[j, :]
        v = jnp.where(ii == pj[None, None, :], 0.0, v)

    l_per_slab = jnp.argmax(v, axis=1)
    m_per_slab = jnp.max(v, axis=1)
    h_star = jnp.argmax(m_per_slab, axis=0)
    ih = jax.lax.broadcasted_iota(jnp.int32, (NH, bsz), 0)
    oh_h = (ih == h_star[None, :]).astype(jnp.float32)
    l_star = jnp.sum(
        l_per_slab.astype(jnp.float32) * oh_h, axis=0).astype(jnp.int32)
    idx = h_star * NL + l_star

    il = jax.lax.broadcasted_iota(jnp.int32, (NL, bsz), 0)
    oh_l = (il == l_star[None, :]).astype(jnp.float32)

    dnr = (dr_ref[...] / (nrm.reshape(NH, 1, NL) + EPS)
           ).reshape(NH * EMBEDDING_DIM, NL)
    c0 = dnr.astype(jnp.bfloat16)
    r1 = dnr - c0.astype(jnp.float32)
    c1 = r1.astype(jnp.bfloat16)
    c2 = (r1 - c1.astype(jnp.float32)).astype(jnp.bfloat16)
    oh_l_b = oh_l.astype(jnp.bfloat16)
    dims = (((1,), (0,)), ((), ()))
    u = (jax.lax.dot_general(c0, oh_l_b, dims,
                             preferred_element_type=jnp.float32)
         + jax.lax.dot_general(c1, oh_l_b, dims,
                               preferred_element_type=jnp.float32)
         + jax.lax.dot_general(c2, oh_l_b, dims,
                               preferred_element_type=jnp.float32))
    d_sel = jnp.sum(
        u.reshape(NH, EMBEDDING_DIM, bsz) * oh_h[:, None, :], axis=0)

    num = jnp.sum(res * d_sel, axis=0, keepdims=True)
    den = jnp.sum(d_sel * d_sel, axis=0, keepdims=True)
    alpha = num / (den + EPS)

    res_out_ref[...] = res - d_sel * alpha
    idx_out_ref[...] = idx[None, :]
    alpha_out_ref[...] = alpha

    if usage_out_ref is None:
        return

    hist = jax.lax.dot_general(
        oh_h, oh_l, (((1,), (1,)), ((), ())),
        precision=jax.lax.Precision.HIGHEST,
        preferred_element_type=jnp.float32)

    @pl.when(t == 0)
    def _init():
        if k > 0:
            usage_out_ref[...] = usage_prev_ref[...] + hist
        else:
            usage_out_ref[...] = hist

    @pl.when(t != 0)
    def _acc():
        usage_out_ref[...] += hist


def _omp_step(k, d_raw, d_raw_r, res, usage_prev, prev_idx):
    b = res.shape[1]
    grid = (b // TILE_B,)
    in_specs = [
        pl.BlockSpec((EMBEDDING_DIM, NUM_EMBEDDINGS), lambda t: (0, 0)),
        pl.BlockSpec((NH, EMBEDDING_DIM, NL), lambda t: (0, 0, 0)),
        pl.BlockSpec((EMBEDDING_DIM, TILE_B), lambda t: (0, t)),
    ]
    args = [d_raw, d_raw_r, res]
    last = k == SPARSITY_LEVEL - 1
    if k > 0:
        in_specs.append(pl.BlockSpec((NH, NL), lambda t: (0, 0)))
        in_specs.append(pl.BlockSpec((k, TILE_B), lambda t: (0, t)))
        args.append(usage_prev)
        args.append(prev_idx)
        if last:
            def body(d_ref, dr_ref, res_ref, up_ref, pi_ref, *out_refs):
                _omp_step_kernel(k, d_ref, dr_ref, res_ref, up_ref, pi_ref,
                                 *out_refs, None)
        else:
            body = functools.partial(_omp_step_kernel, k)
    else:
        def body(d_ref, dr_ref, res_ref, *out_refs):
            _omp_step_kernel(0, d_ref, dr_ref, res_ref, None, None, *out_refs)

    out_shape = [
        jax.ShapeDtypeStruct((EMBEDDING_DIM, b), jnp.float32),
        jax.ShapeDtypeStruct((1, b), jnp.int32),
        jax.ShapeDtypeStruct((1, b), jnp.float32),
    ]
    out_specs = [
        pl.BlockSpec((EMBEDDING_DIM, TILE_B), lambda t: (0, t)),
        pl.BlockSpec((1, TILE_B), lambda t: (0, t)),
        pl.BlockSpec((1, TILE_B), lambda t: (0, t)),
    ]
    if not last:
        out_shape.append(jax.ShapeDtypeStruct((NH, NL), jnp.float32))
        out_specs.append(pl.BlockSpec((NH, NL), lambda t: (0, 0)))
    return pl.pallas_call(
        body,
        grid=grid,
        in_specs=in_specs,
        out_specs=out_specs,
        out_shape=out_shape,
    )(*args)


def _finalize_kernel(d_ref, x_ref, idx_ref, alpha_ref,
                     out_ref, coef_ref, loss_ref):
    t = pl.program_id(0)
    d_raw = d_ref[...]
    nrm = jnp.sqrt(jnp.sum(d_raw * d_raw, axis=0, keepdims=True))
    dn = d_raw / (nrm + EPS)
    x = x_ref[...]
    bsz = x.shape[1]

    ii = jax.lax.broadcasted_iota(jnp.int32, (NUM_EMBEDDINGS, bsz), 0)
    coef = jnp.zeros((NUM_EMBEDDINGS, bsz), jnp.float32)
    for j in range(SPARSITY_LEVEL):
        sel = ii == idx_ref[j, :][None, :]
        coef = jnp.where(sel, alpha_ref[j, :][None, :], coef)
    coef_ref[...] = coef

    z_dl = jax.lax.dot_general(
        dn, coef, (((1,), (0,)), ((), ())),
        preferred_element_type=jnp.float32)
    delta = z_dl - x
    out_ref[...] = x + delta

    part = jnp.sum(delta * delta).reshape(1, 1)

    @pl.when(t == 0)
    def _init():
        loss_ref[...] = part

    @pl.when(t != 0)
    def _acc():
        loss_ref[...] += part


def _finalize(d_raw, x, idx_hist, alpha_hist):
    b = x.shape[1]
    grid = (b // TILE_B,)
    out_shape = [
        jax.ShapeDtypeStruct((EMBEDDING_DIM, b), jnp.float32),
        jax.ShapeDtypeStruct((NUM_EMBEDDINGS, b), jnp.float32),
        jax.ShapeDtypeStruct((1, 1), jnp.float32),
    ]
    return pl.pallas_call(
        _finalize_kernel,
        grid=grid,
        in_specs=[
            pl.BlockSpec((EMBEDDING_DIM, NUM_EMBEDDINGS), lambda t: (0, 0)),
            pl.BlockSpec((EMBEDDING_DIM, TILE_B), lambda t: (0, t)),
            pl.BlockSpec((SPARSITY_LEVEL, TILE_B), lambda t: (0, t)),
            pl.BlockSpec((SPARSITY_LEVEL, TILE_B), lambda t: (0, t)),
        ],
        out_specs=[
            pl.BlockSpec((EMBEDDING_DIM, TILE_B), lambda t: (0, t)),
            pl.BlockSpec((NUM_EMBEDDINGS, TILE_B), lambda t: (0, t)),
            pl.BlockSpec((1, 1), lambda t: (0, 0)),
        ],
        out_shape=out_shape,
    )(d_raw, x, idx_hist, alpha_hist)


def kernel(z_e, dictionary):
    n, c, h, w = z_e.shape
    z = jnp.transpose(z_e, (0, 2, 3, 1))
    x = z.reshape(-1, EMBEDDING_DIM).T

    d_raw_r = dictionary.reshape(EMBEDDING_DIM, NH, NL).transpose(1, 0, 2)

    res = x
    usage = None
    idx_list = []
    alpha_list = []
    for k in range(SPARSITY_LEVEL):
        prev_idx = jnp.concatenate(idx_list, axis=0) if k > 0 else None
        outs = _omp_step(k, dictionary, d_raw_r, res, usage, prev_idx)
        if k < SPARSITY_LEVEL - 1:
            res, idx_k, alpha_k, usage = outs
        else:
            res, idx_k, alpha_k = outs
        idx_list.append(idx_k)
        alpha_list.append(alpha_k)

    idx_hist = jnp.concatenate(idx_list, axis=0)
    alpha_hist = jnp.concatenate(alpha_list, axis=0)

    z_dl_st_flat, coef, loss_sum = _finalize(dictionary, x, idx_hist,
                                             alpha_hist)

    m = loss_sum[0, 0] / (n * h * w * EMBEDDING_DIM)
    loss = COMMITMENT_COST * m + m

    out1 = z_dl_st_flat.T.reshape(n, h, w, c).transpose(0, 3, 1, 2)
    return (out1, loss, coef)
